# baseline (speedup 1.0000x reference)
import functools

import jax
import jax.numpy as jnp
from jax import lax
from jax.experimental import pallas as pl
from jax.experimental.pallas import tpu as pltpu
from jax.experimental.pallas import tpu_sc as plsc

MAXP = 100000
EMB = 64
LANES = 16
B = 4096
S = 200
NW = 32
BL = B // NW
DC = EMB // LANES


def _make_kernel():
    info = plsc.get_sparse_core_info()
    assert info.num_cores * info.num_subcores == NW

    mesh = plsc.VectorSubcoreMesh(core_axis_name="c", subcore_axis_name="s")

    @functools.partial(
        pl.kernel,
        mesh=mesh,
        compiler_params=pltpu.CompilerParams(
            use_tc_tiling_on_sc=False, needs_layout_passes=False
        ),
        out_type=jax.ShapeDtypeStruct((S, EMB // 8, NW, 8, BL), jnp.float32),
        scratch_types=[
            pltpu.VMEM((S, BL), jnp.int32),
            pltpu.VMEM((BL,), jnp.int32),
            pltpu.VMEM((BL,), jnp.int32),
            pltpu.VMEM((BL, EMB), jnp.float32),
            pltpu.VMEM((BL, EMB), jnp.float32),
            pltpu.VMEM((EMB // 8, 8, BL), jnp.float32),
            pltpu.VMEM((EMB // 8, 8, BL), jnp.float32),
            pltpu.SemaphoreType.DMA,
            pltpu.SemaphoreType.DMA,
            pltpu.SemaphoreType.DMA,
            pltpu.SemaphoreType.DMA,
            pltpu.SemaphoreType.DMA,
        ],
    )
    def k(xt_hbm, tab_hbm, out_hbm, slab, gv0, gv1, r0, r1, t0, t1,
          isem, g0, g1, o0, o1):
        wid = lax.axis_index("s") * info.num_cores + lax.axis_index("c")
        gv = (gv0, gv1)
        rows = (r0, r1)
        tv = (t0, t1)
        gsem = (g0, g1)
        osem = (o0, o1)

        pltpu.async_copy(
            xt_hbm.at[:, pl.ds(wid * BL, BL)], slab, isem
        ).wait()

        iota = lax.iota(jnp.int32, LANES)
        perms = [(iota + j) & 15 for j in range(LANES)]
        cols = [iota + d16 * LANES for d16 in range(DC)]
        d0s = [(iota + d16 * LANES) >> 3 for d16 in range(DC)]
        dss = iota & 7

        def build_gvec(s, g):
            vs = [
                slab[s, pl.ds(c * LANES, LANES)] for c in range(BL // LANES)
            ]
            vs = [
                jnp.minimum(jnp.maximum(v, -MAXP), MAXP) + MAXP for v in vs
            ]
            for c, v in enumerate(vs):
                g[pl.ds(c * LANES, LANES)] = v

        def transpose(r, t):
            def tbody(b16, carry):
                b0 = b16 * LANES
                ridxs = [perms[j] + b0 for j in range(LANES)]
                for d16 in range(DC):
                    vs = [
                        plsc.load_gather(r, [ridxs[j], cols[d16]])
                        for j in range(LANES)
                    ]
                    for j in range(LANES):
                        plsc.store_scatter(
                            t, [d0s[d16], dss, ridxs[j]], vs[j]
                        )
                return carry

            lax.fori_loop(0, BL // LANES, tbody, 0)

        def drain_gather(p):
            pltpu.make_async_copy(
                tab_hbm.at[pl.ds(0, BL)], rows[p], gsem[p]
            ).wait()

        def store(s, p):
            pltpu.async_copy(tv[p], out_hbm.at[s, :, wid, :, :], osem[p])

        def drain_store(p):
            pltpu.make_async_copy(
                tv[p], out_hbm.at[0, :, wid, :, :], osem[p]
            ).wait()

        def body(s2, carry):
            for p in range(2):
                s = s2 * 2 + p
                np_ = 1 - p

                @pl.when(s >= 2)
                def _():
                    drain_store(p)

                build_gvec(s, gv[p])
                pltpu.async_copy(tab_hbm.at[gv[p]], rows[p], gsem[p])

                @pl.when(s >= 1)
                def _():
                    drain_gather(np_)
                    transpose(rows[np_], tv[np_])
                    store(s - 1, np_)

            return carry

        lax.fori_loop(0, S // 2, body, 0)

        drain_gather(1)
        transpose(rows[1], tv[1])
        store(S - 1, 1)
        drain_store(0)
        drain_store(1)

    return k


def kernel(x, weights):
    assert x.shape == (B, S)
    o = _make_kernel()(x.T, weights)
    return jnp.transpose(o, (2, 4, 0, 1, 3)).reshape(B, S, EMB)


# speedup vs baseline: 1.0908x; 1.0908x over previous
import functools

import jax
import jax.numpy as jnp
from jax import lax
from jax.experimental import pallas as pl
from jax.experimental.pallas import tpu as pltpu
from jax.experimental.pallas import tpu_sc as plsc

MAXP = 100000
EMB = 64
LANES = 16
B = 4096
S = 200
NW = 32
BL = B // NW
DC = EMB // LANES
SP = 2
NPH = S // SP


def _make_kernel():
    info = plsc.get_sparse_core_info()
    assert info.num_cores * info.num_subcores == NW

    mesh = plsc.VectorSubcoreMesh(core_axis_name="c", subcore_axis_name="s")

    @functools.partial(
        pl.kernel,
        mesh=mesh,
        compiler_params=pltpu.CompilerParams(
            use_tc_tiling_on_sc=False, needs_layout_passes=False
        ),
        out_type=jax.ShapeDtypeStruct((S * EMB // 8, NW, 8, BL), jnp.float32),
        scratch_types=[
            pltpu.VMEM((S, BL), jnp.int32),
            pltpu.VMEM((SP, BL), jnp.int32),
            pltpu.VMEM((SP, BL), jnp.int32),
            pltpu.VMEM((SP * BL, EMB), jnp.float32),
            pltpu.VMEM((SP * BL, EMB), jnp.float32),
            pltpu.VMEM((SP * EMB // 8, 8, BL), jnp.float32),
            pltpu.VMEM((SP * EMB // 8, 8, BL), jnp.float32),
            pltpu.SemaphoreType.DMA,
            pltpu.SemaphoreType.DMA,
            pltpu.SemaphoreType.DMA,
            pltpu.SemaphoreType.DMA,
            pltpu.SemaphoreType.DMA,
        ],
    )
    def k(xt_hbm, tab_hbm, out_hbm, slab, gv0, gv1, r0, r1, t0, t1,
          isem, g0, g1, o0, o1):
        wid = lax.axis_index("s") * info.num_cores + lax.axis_index("c")
        gv = (gv0, gv1)
        rows = (r0, r1)
        tv = (t0, t1)
        gsem = (g0, g1)
        osem = (o0, o1)

        pltpu.async_copy(
            xt_hbm.at[:, pl.ds(wid * BL, BL)], slab, isem
        ).wait()

        iota = lax.iota(jnp.int32, LANES)
        perms = [(iota + j) & 15 for j in range(LANES)]
        cols = [iota + d16 * LANES for d16 in range(DC)]
        d0s = [(iota + d16 * LANES) >> 3 for d16 in range(DC)]
        dss = iota & 7

        def build_gvec(s, g):
            vs = [
                slab[s + sp, pl.ds(c * LANES, LANES)]
                for sp in range(SP)
                for c in range(BL // LANES)
            ]
            vs = [
                jnp.minimum(jnp.maximum(v, -MAXP), MAXP) + MAXP for v in vs
            ]
            i = 0
            for sp in range(SP):
                for c in range(BL // LANES):
                    g[sp, pl.ds(c * LANES, LANES)] = vs[i]
                    i += 1

        def transpose(r, t):
            def tbody(b16, carry):
                b0 = b16 * LANES
                sd = (b16 & 8) << 0
                sd0 = [d0s[d16] + sd for d16 in range(DC)]
                ridxs = [perms[j] + b0 for j in range(LANES)]
                bcols = [x & 127 for x in ridxs]
                for d16 in range(DC):
                    vs = [
                        plsc.load_gather(r, [ridxs[j], cols[d16]])
                        for j in range(LANES)
                    ]
                    for j in range(LANES):
                        plsc.store_scatter(
                            t, [sd0[d16], dss, bcols[j]], vs[j]
                        )
                return carry

            lax.fori_loop(0, SP * BL // LANES, tbody, 0)

        def drain_gather(p):
            pltpu.make_async_copy(
                tab_hbm.at[pl.ds(0, SP * BL)], rows[p], gsem[p]
            ).wait()

        def store(ss, p):
            pltpu.async_copy(
                tv[p],
                out_hbm.at[pl.ds(ss * SP * EMB // 8, SP * EMB // 8), wid, :, :],
                osem[p],
            )

        def drain_store(p):
            pltpu.make_async_copy(
                tv[p],
                out_hbm.at[pl.ds(0, SP * EMB // 8), wid, :, :],
                osem[p],
            ).wait()

        def body(g2, carry):
            for p in range(2):
                ss = g2 * 2 + p
                s = ss * SP
                np_ = 1 - p

                @pl.when(ss >= 2)
                def _():
                    drain_store(p)

                build_gvec(s, gv[p])
                for sp in range(SP):
                    pltpu.async_copy(
                        tab_hbm.at[gv[p].at[sp]],
                        rows[p].at[pl.ds(sp * BL, BL)],
                        gsem[p],
                    )

                @pl.when(ss >= 1)
                def _():
                    drain_gather(np_)
                    transpose(rows[np_], tv[np_])
                    store(ss - 1, np_)

            return carry

        lax.fori_loop(0, NPH // 2, body, 0)

        drain_gather(1)
        transpose(rows[1], tv[1])
        store(NPH - 1, 1)
        drain_store(0)
        drain_store(1)

    return k


def kernel(x, weights):
    assert x.shape == (B, S)
    o = _make_kernel()(x.T, weights)
    o5 = o.reshape(S, EMB // 8, NW, 8, BL)
    return jnp.transpose(o5, (2, 4, 0, 1, 3)).reshape(B, S, EMB)


# speedup vs baseline: 1.1392x; 1.0443x over previous
import functools

import jax
import jax.numpy as jnp
from jax import lax
from jax.experimental import pallas as pl
from jax.experimental.pallas import tpu as pltpu
from jax.experimental.pallas import tpu_sc as plsc

MAXP = 100000
EMB = 64
LANES = 16
B = 4096
S = 200
NW = 32
BL = B // NW
DC = EMB // LANES
SP = 2
NPH = S // SP


def _make_kernel():
    info = plsc.get_sparse_core_info()
    assert info.num_cores * info.num_subcores == NW

    mesh = plsc.VectorSubcoreMesh(core_axis_name="c", subcore_axis_name="s")

    @functools.partial(
        pl.kernel,
        mesh=mesh,
        compiler_params=pltpu.CompilerParams(
            use_tc_tiling_on_sc=False, needs_layout_passes=False
        ),
        out_type=jax.ShapeDtypeStruct((S * EMB // 8, NW, 8, BL), jnp.float32),
        scratch_types=[
            pltpu.VMEM((S, BL), jnp.int32),
            pltpu.VMEM((SP, BL), jnp.int32),
            pltpu.VMEM((SP, BL), jnp.int32),
            pltpu.VMEM((SP * BL, EMB), jnp.float32),
            pltpu.VMEM((SP * BL, EMB), jnp.float32),
            pltpu.VMEM((SP * EMB // 8, 8, BL), jnp.float32),
            pltpu.VMEM((SP * EMB // 8, 8, BL), jnp.float32),
            pltpu.SemaphoreType.DMA,
            pltpu.SemaphoreType.DMA,
            pltpu.SemaphoreType.DMA,
            pltpu.SemaphoreType.DMA,
            pltpu.SemaphoreType.DMA,
        ],
    )
    def k(xt_hbm, tab_hbm, out_hbm, slab, gv0, gv1, r0, r1, t0, t1,
          isem, g0, g1, o0, o1):
        wid = lax.axis_index("s") * info.num_cores + lax.axis_index("c")
        gv = (gv0, gv1)
        rows = (r0, r1)
        tv = (t0, t1)
        gsem = (g0, g1)
        osem = (o0, o1)

        pltpu.async_copy(
            xt_hbm.at[:, pl.ds(wid * BL, BL)], slab, isem
        ).wait()

        iota = lax.iota(jnp.int32, LANES)
        perms = [(iota + j) & 15 for j in range(LANES)]
        cols = [iota + d16 * LANES for d16 in range(DC)]
        d0s = [(iota + d16 * LANES) >> 3 for d16 in range(DC)]
        dss = iota & 7

        def build_gvec(s, g):
            vs = [
                slab[s + sp, pl.ds(c * LANES, LANES)]
                for sp in range(SP)
                for c in range(BL // LANES)
            ]
            vs = [
                (jnp.minimum(jnp.maximum(v, -MAXP), MAXP) + MAXP) << 1
                for v in vs
            ]
            i = 0
            for sp in range(SP):
                for c in range(BL // LANES):
                    g[sp, pl.ds(c * LANES, LANES)] = vs[i]
                    i += 1

        def transpose(r, t):
            def tbody(b16, carry):
                b0 = b16 * LANES
                sd = (b16 & 8) << 0
                sd0 = [d0s[d16] + sd for d16 in range(DC)]
                ridxs = [perms[j] + b0 for j in range(LANES)]
                bcols = [x & 127 for x in ridxs]
                for d16 in range(DC):
                    vs = [
                        plsc.load_gather(r, [ridxs[j], cols[d16]])
                        for j in range(LANES)
                    ]
                    for j in range(LANES):
                        plsc.store_scatter(
                            t, [sd0[d16], dss, bcols[j]], vs[j]
                        )
                return carry

            lax.fori_loop(0, SP * BL // LANES, tbody, 0)

        def drain_gather(p):
            pltpu.make_async_copy(
                tab_hbm.at[pl.ds(0, SP * BL)], rows[p], gsem[p]
            ).wait()

        def store(ss, p):
            pltpu.async_copy(
                tv[p],
                out_hbm.at[pl.ds(ss * SP * EMB // 8, SP * EMB // 8), wid, :, :],
                osem[p],
            )

        def drain_store(p):
            pltpu.make_async_copy(
                tv[p],
                out_hbm.at[pl.ds(0, SP * EMB // 8), wid, :, :],
                osem[p],
            ).wait()

        def body(g2, carry):
            for p in range(2):
                ss = g2 * 2 + p
                s = ss * SP
                np_ = 1 - p

                @pl.when(ss >= 2)
                def _():
                    drain_store(p)

                build_gvec(s, gv[p])
                for sp in range(SP):
                    pltpu.async_copy(
                        tab_hbm.at[gv[p].at[sp]],
                        rows[p].at[pl.ds(sp * BL, BL)],
                        gsem[p],
                    )

                @pl.when(ss >= 1)
                def _():
                    drain_gather(np_)
                    transpose(rows[np_], tv[np_])
                    store(ss - 1, np_)

            return carry

        lax.fori_loop(0, NPH // 2, body, 0)

        drain_gather(1)
        transpose(rows[1], tv[1])
        store(NPH - 1, 1)
        drain_store(0)
        drain_store(1)

    return k


def kernel(x, weights):
    assert x.shape == (B, S)
    wpad = jnp.pad(weights, ((0, 7), (0, EMB)))
    w2 = wpad.reshape(2 * 200008, EMB)
    o = _make_kernel()(x.T, w2)
    o5 = o.reshape(S, EMB // 8, NW, 8, BL)
    return jnp.transpose(o5, (2, 4, 0, 1, 3)).reshape(B, S, EMB)


# speedup vs baseline: 1.1600x; 1.0183x over previous
import functools

import jax
import jax.numpy as jnp
from jax import lax
from jax.experimental import pallas as pl
from jax.experimental.pallas import tpu as pltpu
from jax.experimental.pallas import tpu_sc as plsc

MAXP = 100000
EMB = 64
LANES = 16
B = 4096
S = 200
NW = 32
BL = B // NW
DC = EMB // LANES
SP = 2
NPH = S // SP


def _make_kernel():
    info = plsc.get_sparse_core_info()
    assert info.num_cores * info.num_subcores == NW

    mesh = plsc.VectorSubcoreMesh(core_axis_name="c", subcore_axis_name="s")

    @functools.partial(
        pl.kernel,
        mesh=mesh,
        compiler_params=pltpu.CompilerParams(
            use_tc_tiling_on_sc=False, needs_layout_passes=False
        ),
        out_type=jax.ShapeDtypeStruct((S * EMB // 8, NW, 8, BL), jnp.float32),
        scratch_types=[
            pltpu.VMEM((S, BL), jnp.int32),
            pltpu.VMEM((SP, BL), jnp.int32),
            pltpu.VMEM((SP, BL), jnp.int32),
            pltpu.VMEM((SP * BL, EMB), jnp.float32),
            pltpu.VMEM((SP * BL, EMB), jnp.float32),
            pltpu.VMEM((SP * EMB // 8, 8, BL), jnp.float32),
            pltpu.VMEM((SP * EMB // 8, 8, BL), jnp.float32),
            pltpu.SemaphoreType.DMA,
            pltpu.SemaphoreType.DMA,
            pltpu.SemaphoreType.DMA,
            pltpu.SemaphoreType.DMA,
            pltpu.SemaphoreType.DMA,
        ],
    )
    def k(xt_hbm, tab_hbm, out_hbm, slab, gv0, gv1, r0, r1, t0, t1,
          isem, g0, g1, o0, o1):
        wid = lax.axis_index("s") * info.num_cores + lax.axis_index("c")
        gv = (gv0, gv1)
        rows = (r0, r1)
        tv = (t0, t1)
        gsem = (g0, g1)
        osem = (o0, o1)

        pltpu.async_copy(
            xt_hbm.at[:, pl.ds(wid * BL, BL)], slab, isem
        ).wait()

        iota = lax.iota(jnp.int32, LANES)
        perms = [(iota + j) & 15 for j in range(LANES)]
        cols = [iota + d16 * LANES for d16 in range(DC)]
        d0s = [(iota + d16 * LANES) >> 3 for d16 in range(DC)]
        dss = iota & 7

        def build_gvec(s, g):
            vs = [
                slab[s + sp, pl.ds(c * LANES, LANES)]
                for sp in range(SP)
                for c in range(BL // LANES)
            ]
            vs = [
                (jnp.minimum(jnp.maximum(v, -MAXP), MAXP) + MAXP) << 1
                for v in vs
            ]
            i = 0
            for sp in range(SP):
                for c in range(BL // LANES):
                    g[sp, pl.ds(c * LANES, LANES)] = vs[i]
                    i += 1

        def transpose(r, t):
            def tbody(i, carry):
                for k in range(4):
                    b16 = i * 4 + k
                    b0 = b16 * LANES
                    sd0 = [d0s[d16] + (b16 & 8) for d16 in range(DC)]
                    ridxs = [perms[j] + b0 for j in range(LANES)]
                    bcols = [x & 127 for x in ridxs]
                    for d16 in range(DC):
                        vs = [
                            plsc.load_gather(r, [ridxs[j], cols[d16]])
                            for j in range(LANES)
                        ]
                        for j in range(LANES):
                            plsc.store_scatter(
                                t, [sd0[d16], dss, bcols[j]], vs[j]
                            )
                return carry

            lax.fori_loop(0, SP * BL // LANES // 4, tbody, 0)

        def drain_gather(p):
            pltpu.make_async_copy(
                tab_hbm.at[pl.ds(0, SP * BL)], rows[p], gsem[p]
            ).wait()

        def store(ss, p):
            pltpu.async_copy(
                tv[p],
                out_hbm.at[pl.ds(ss * SP * EMB // 8, SP * EMB // 8), wid, :, :],
                osem[p],
            )

        def drain_store(p):
            pltpu.make_async_copy(
                tv[p],
                out_hbm.at[pl.ds(0, SP * EMB // 8), wid, :, :],
                osem[p],
            ).wait()

        def body(g2, carry):
            for p in range(2):
                ss = g2 * 2 + p
                s = ss * SP
                np_ = 1 - p

                @pl.when(ss >= 2)
                def _():
                    drain_store(p)

                build_gvec(s, gv[p])
                for sp in range(SP):
                    pltpu.async_copy(
                        tab_hbm.at[gv[p].at[sp]],
                        rows[p].at[pl.ds(sp * BL, BL)],
                        gsem[p],
                    )

                @pl.when(ss >= 1)
                def _():
                    drain_gather(np_)
                    transpose(rows[np_], tv[np_])
                    store(ss - 1, np_)

            return carry

        lax.fori_loop(0, NPH // 2, body, 0)

        drain_gather(1)
        transpose(rows[1], tv[1])
        store(NPH - 1, 1)
        drain_store(0)
        drain_store(1)

    return k


def kernel(x, weights):
    assert x.shape == (B, S)
    wpad = jnp.pad(weights, ((0, 7), (0, EMB)))
    w2 = wpad.reshape(2 * 200008, EMB)
    o = _make_kernel()(x.T, w2)
    o5 = o.reshape(S, EMB // 8, NW, 8, BL)
    return jnp.transpose(o5, (2, 4, 0, 1, 3)).reshape(B, S, EMB)
